# baseline (speedup 1.0000x reference)
import functools

import jax
import jax.numpy as jnp
from jax import lax
from jax.experimental import pallas as pl
from jax.experimental.pallas import tpu as pltpu
from jax.experimental.pallas import tpu_sc as plsc

L = 16
NC = 2
NS = 16
NW = NC * NS
B = 16384
K = 16
BPW = B // NW
NBLK = BPW // L
NROW = 1000000
MVN = 163840
GRID = (NROW + MVN - 1) // MVN


def _mv_body(wt_ref, ht_ref, wu_ref, wv_ref, b_ref, a_ref, c_ref):
    a_ref[...] = jnp.sum(wt_ref[...] * wu_ref[...], axis=0) + b_ref[0, 0]
    c_ref[...] = jnp.sum(ht_ref[...] * wv_ref[...], axis=0)


_mv_call = pl.pallas_call(
    _mv_body,
    grid=(GRID,),
    in_specs=[
        pl.BlockSpec((K, MVN), lambda i: (0, i)),
        pl.BlockSpec((K, MVN), lambda i: (0, i)),
        pl.BlockSpec((K, 1), lambda i: (0, 0)),
        pl.BlockSpec((K, 1), lambda i: (0, 0)),
        pl.BlockSpec((1, 1), lambda i: (0, 0)),
    ],
    out_specs=[
        pl.BlockSpec((MVN,), lambda i: (i,)),
        pl.BlockSpec((MVN,), lambda i: (i,)),
    ],
    out_shape=[
        jax.ShapeDtypeStruct((NROW,), jnp.float32),
        jax.ShapeDtypeStruct((NROW,), jnp.float32),
    ],
)


def _sc_body(u_hbm, v_hbm, a_hbm, c_hbm, out_hbm,
             uidx_v, vidx_v, av, cv, out_v, sem_a, sem_c):
    wid = lax.axis_index("s") * NC + lax.axis_index("c")
    base = wid * BPW

    pltpu.sync_copy(u_hbm.at[pl.ds(base, BPW)], uidx_v)
    pltpu.sync_copy(v_hbm.at[pl.ds(base, BPW)], vidx_v)
    ca = pltpu.async_copy(a_hbm.at[uidx_v], av, sem_a)
    cc = pltpu.async_copy(c_hbm.at[vidx_v], cv, sem_c)
    ca.wait()
    cc.wait()

    def block(i, _):
        z = av[pl.ds(i * L, L)] + cv[pl.ds(i * L, L)]
        out_v[pl.ds(i * L, L)] = 1.0 / (1.0 + jnp.exp(-z))
        return _

    lax.fori_loop(0, NBLK, block, None)
    pltpu.sync_copy(out_v, out_hbm.at[pl.ds(base, BPW)])


@functools.partial(
    pl.kernel,
    out_type=jax.ShapeDtypeStruct((B,), jnp.float32),
    mesh=plsc.VectorSubcoreMesh(core_axis_name="c", subcore_axis_name="s"),
    compiler_params=pltpu.CompilerParams(
        use_tc_tiling_on_sc=False, needs_layout_passes=False),
    scratch_types=[
        pltpu.VMEM((BPW,), jnp.int32),
        pltpu.VMEM((BPW,), jnp.int32),
        pltpu.VMEM((BPW,), jnp.float32),
        pltpu.VMEM((BPW,), jnp.float32),
        pltpu.VMEM((BPW,), jnp.float32),
        pltpu.SemaphoreType.DMA,
        pltpu.SemaphoreType.DMA,
    ],
)
def _sc_lookup(*refs):
    _sc_body(*refs)


def kernel(x, W, H, lin_w, lin_b):
    u_idx = x[:, 0]
    v_idx = x[:, 1]
    wu = lin_w[:, :K].reshape(K, 1)
    wv = lin_w[:, K:].reshape(K, 1)
    bias = lin_b.reshape(1, 1)
    a, c = _mv_call(W.T, H.T, wu, wv, bias)
    return _sc_lookup(u_idx, v_idx, a, c)


# speedup vs baseline: 1.0506x; 1.0506x over previous
import functools

import jax
import jax.numpy as jnp
from jax import lax
from jax.experimental import pallas as pl
from jax.experimental.pallas import tpu as pltpu
from jax.experimental.pallas import tpu_sc as plsc

L = 16
NC = 2
NS = 16
NW = NC * NS
B = 16384
K = 16
BPW = B // NW
NBLK = BPW // L
NROW = 1000000
MVN = 131072
GRID = (NROW + MVN - 1) // MVN


def _mv_body(wt_ref, ht_ref, wu_ref, wv_ref, b_ref, a_ref, c_ref):
    a_ref[...] = jnp.sum(wt_ref[...] * wu_ref[...], axis=0) + b_ref[0, 0]
    c_ref[...] = jnp.sum(ht_ref[...] * wv_ref[...], axis=0)


_mv_call = pl.pallas_call(
    _mv_body,
    grid=(GRID,),
    in_specs=[
        pl.BlockSpec((K, MVN), lambda i: (0, i)),
        pl.BlockSpec((K, MVN), lambda i: (0, i)),
        pl.BlockSpec((K, 1), lambda i: (0, 0)),
        pl.BlockSpec((K, 1), lambda i: (0, 0)),
        pl.BlockSpec((1, 1), lambda i: (0, 0)),
    ],
    out_specs=[
        pl.BlockSpec((MVN,), lambda i: (i,)),
        pl.BlockSpec((MVN,), lambda i: (i,)),
    ],
    out_shape=[
        jax.ShapeDtypeStruct((NROW,), jnp.float32),
        jax.ShapeDtypeStruct((NROW,), jnp.float32),
    ],
)


def _sc_body(u_hbm, v_hbm, a_hbm, c_hbm, out_hbm,
             uidx_v, vidx_v, av, cv, out_v, sem_a, sem_c):
    wid = lax.axis_index("s") * NC + lax.axis_index("c")
    base = wid * BPW

    pltpu.sync_copy(u_hbm.at[pl.ds(base, BPW)], uidx_v)
    pltpu.sync_copy(v_hbm.at[pl.ds(base, BPW)], vidx_v)
    ca = pltpu.async_copy(a_hbm.at[uidx_v], av, sem_a)
    cc = pltpu.async_copy(c_hbm.at[vidx_v], cv, sem_c)
    ca.wait()
    cc.wait()

    def block(i, _):
        z = av[pl.ds(i * L, L)] + cv[pl.ds(i * L, L)]
        out_v[pl.ds(i * L, L)] = 1.0 / (1.0 + jnp.exp(-z))
        return _

    lax.fori_loop(0, NBLK, block, None)
    pltpu.sync_copy(out_v, out_hbm.at[pl.ds(base, BPW)])


@functools.partial(
    pl.kernel,
    out_type=jax.ShapeDtypeStruct((B,), jnp.float32),
    mesh=plsc.VectorSubcoreMesh(core_axis_name="c", subcore_axis_name="s"),
    compiler_params=pltpu.CompilerParams(
        use_tc_tiling_on_sc=False, needs_layout_passes=False),
    scratch_types=[
        pltpu.VMEM((BPW,), jnp.int32),
        pltpu.VMEM((BPW,), jnp.int32),
        pltpu.VMEM((BPW,), jnp.float32),
        pltpu.VMEM((BPW,), jnp.float32),
        pltpu.VMEM((BPW,), jnp.float32),
        pltpu.SemaphoreType.DMA,
        pltpu.SemaphoreType.DMA,
    ],
)
def _sc_lookup(*refs):
    _sc_body(*refs)


def kernel(x, W, H, lin_w, lin_b):
    u_idx = x[:, 0]
    v_idx = x[:, 1]
    wu = lin_w[:, :K].reshape(K, 1)
    wv = lin_w[:, K:].reshape(K, 1)
    bias = lin_b.reshape(1, 1)
    a, c = _mv_call(W.T, H.T, wu, wv, bias)
    return _sc_lookup(u_idx, v_idx, a, c)


# speedup vs baseline: 1.0631x; 1.0119x over previous
import functools

import jax
import jax.numpy as jnp
from jax import lax
from jax.experimental import pallas as pl
from jax.experimental.pallas import tpu as pltpu
from jax.experimental.pallas import tpu_sc as plsc

L = 16
NC = 2
NS = 16
NW = NC * NS
B = 16384
K = 16
BPW = B // NW
NBLK = BPW // L
NROW = 1000000
MVN = 131072
GRID = (NROW + MVN - 1) // MVN
NROWP = GRID * MVN


def _mv_body(wt_ref, ht_ref, wu_ref, wv_ref, b_ref, a_hbm, c_hbm,
             abuf, cbuf, sem_ab, sem_cb):
    i = pl.program_id(0)
    slot = lax.rem(i, 2)

    @pl.when(i >= 2)
    def _():
        pltpu.make_async_copy(abuf.at[slot], a_hbm.at[pl.ds(0, MVN)],
                              sem_ab.at[slot]).wait()
        pltpu.make_async_copy(cbuf.at[slot], c_hbm.at[pl.ds(0, MVN)],
                              sem_cb.at[slot]).wait()

    abuf[slot, :] = jnp.sum(wt_ref[...] * wu_ref[...], axis=0) + b_ref[0, 0]
    cbuf[slot, :] = jnp.sum(ht_ref[...] * wv_ref[...], axis=0)
    pltpu.async_copy(abuf.at[slot], a_hbm.at[pl.ds(i * MVN, MVN)],
                     sem_ab.at[slot])
    pltpu.async_copy(cbuf.at[slot], c_hbm.at[pl.ds(i * MVN, MVN)],
                     sem_cb.at[slot])

    @pl.when(i == GRID - 1)
    def _():
        for s in range(2):
            pltpu.make_async_copy(abuf.at[s], a_hbm.at[pl.ds(0, MVN)],
                                  sem_ab.at[s]).wait()
            pltpu.make_async_copy(cbuf.at[s], c_hbm.at[pl.ds(0, MVN)],
                                  sem_cb.at[s]).wait()


_mv_call = pl.pallas_call(
    _mv_body,
    grid=(GRID,),
    in_specs=[
        pl.BlockSpec((K, MVN), lambda i: (0, i)),
        pl.BlockSpec((K, MVN), lambda i: (0, i)),
        pl.BlockSpec((K, 1), lambda i: (0, 0)),
        pl.BlockSpec((K, 1), lambda i: (0, 0)),
        pl.BlockSpec((1, 1), lambda i: (0, 0)),
    ],
    out_specs=[
        pl.BlockSpec(memory_space=pl.ANY),
        pl.BlockSpec(memory_space=pl.ANY),
    ],
    out_shape=[
        jax.ShapeDtypeStruct((NROWP,), jnp.float32),
        jax.ShapeDtypeStruct((NROWP,), jnp.float32),
    ],
    scratch_shapes=[
        pltpu.VMEM((2, MVN), jnp.float32),
        pltpu.VMEM((2, MVN), jnp.float32),
        pltpu.SemaphoreType.DMA((2,)),
        pltpu.SemaphoreType.DMA((2,)),
    ],
)


def _sc_body(u_hbm, v_hbm, a_hbm, c_hbm, out_hbm,
             uidx_v, vidx_v, av, cv, out_v, sem_a, sem_c):
    wid = lax.axis_index("s") * NC + lax.axis_index("c")
    base = wid * BPW

    pltpu.sync_copy(u_hbm.at[pl.ds(base, BPW)], uidx_v)
    pltpu.sync_copy(v_hbm.at[pl.ds(base, BPW)], vidx_v)
    ca = pltpu.async_copy(a_hbm.at[uidx_v], av, sem_a)
    cc = pltpu.async_copy(c_hbm.at[vidx_v], cv, sem_c)
    ca.wait()
    cc.wait()

    def block(i, _):
        z = av[pl.ds(i * L, L)] + cv[pl.ds(i * L, L)]
        out_v[pl.ds(i * L, L)] = 1.0 / (1.0 + jnp.exp(-z))
        return _

    lax.fori_loop(0, NBLK, block, None)
    pltpu.sync_copy(out_v, out_hbm.at[pl.ds(base, BPW)])


@functools.partial(
    pl.kernel,
    out_type=jax.ShapeDtypeStruct((B,), jnp.float32),
    mesh=plsc.VectorSubcoreMesh(core_axis_name="c", subcore_axis_name="s"),
    compiler_params=pltpu.CompilerParams(
        use_tc_tiling_on_sc=False, needs_layout_passes=False),
    scratch_types=[
        pltpu.VMEM((BPW,), jnp.int32),
        pltpu.VMEM((BPW,), jnp.int32),
        pltpu.VMEM((BPW,), jnp.float32),
        pltpu.VMEM((BPW,), jnp.float32),
        pltpu.VMEM((BPW,), jnp.float32),
        pltpu.SemaphoreType.DMA,
        pltpu.SemaphoreType.DMA,
    ],
)
def _sc_lookup(*refs):
    _sc_body(*refs)


def kernel(x, W, H, lin_w, lin_b):
    u_idx = x[:, 0]
    v_idx = x[:, 1]
    wu = lin_w[:, :K].reshape(K, 1)
    wv = lin_w[:, K:].reshape(K, 1)
    bias = lin_b.reshape(1, 1)
    a, c = _mv_call(W.T, H.T, wu, wv, bias)
    return _sc_lookup(u_idx, v_idx, a, c)
